# baseline (speedup 1.0000x reference)
import functools

import jax
import jax.numpy as jnp
from jax import lax
from jax.experimental import pallas as pl
from jax.experimental.pallas import tpu as pltpu
from jax.experimental.pallas import tpu_sc as plsc

B = 16384
F = 26
V1 = 51
D = 32

NC = 2
NS = 16
L = 16
NW = NC * NS

BPW = B // NW
DP = D // 2
DT = D // 8
BT = B // 128
BTW = BPW // 128
TT = F * DP * V1

_mesh = plsc.VectorSubcoreMesh(core_axis_name="c", subcore_axis_name="s")

_HI_MASK = -65536


@functools.partial(
    pl.kernel,
    mesh=_mesh,
    out_type=jax.ShapeDtypeStruct((F, DT, BT, 8, 128), jnp.float32),
    compiler_params=pltpu.CompilerParams(use_tc_tiling_on_sc=False,
                                         needs_layout_passes=False),
    scratch_types=[
        pltpu.VMEM((TT,), jnp.int32),
        pltpu.VMEM((BPW,), jnp.int32),
        pltpu.VMEM((BPW,), jnp.int32),
        pltpu.VMEM((DT, BTW, 8, 128), jnp.float32),
        pltpu.VMEM((DT, BTW, 8, 128), jnp.float32),
        pltpu.SemaphoreType.DMA,
        pltpu.SemaphoreType.DMA,
        pltpu.SemaphoreType.DMA,
        pltpu.SemaphoreType.DMA,
    ],
)
def _gather_kernel(idx_hbm, tt_hbm, out_hbm, tt_v, idx0, idx1, blk0, blk1,
                   osem0, osem1, isem0, isem1):
    wid = lax.axis_index("s") * NC + lax.axis_index("c")

    idxb = (idx0, idx1)
    isems = (isem0, isem1)
    blks = (blk0, blk1)
    sems = (osem0, osem1)

    def stage_idx(f, b):
        pltpu.async_copy(idx_hbm.at[f, wid], idxb[b], isems[b])

    def wait_idx(b):
        pltpu.make_async_copy(idx_hbm.at[0, wid], idxb[b], isems[b]).wait()

    stage_idx(0, 0)
    stage_idx(1, 1)
    pltpu.sync_copy(tt_hbm, tt_v)

    def fill_f(f, blk, idx_v):
        base0 = f * (DP * V1)

        @plsc.parallel_loop(0, BTW * 8, unroll=1)
        def _grp(k):
            bt = k // 8
            j = lax.rem(k, 8)
            idxv = idx_v[pl.ds(bt * 128 + j * L, L)]
            for dp in range(DP):
                base = base0 + dp * V1
                raw = plsc.load_gather(tt_v, [idxv + base])
                even = plsc.bitcast(raw << 16, jnp.float32)
                odd = plsc.bitcast(raw & _HI_MASK, jnp.float32)
                d0 = 2 * dp
                blk[d0 // 8, bt, d0 % 8, pl.ds(j * L, L)] = even
                blk[d0 // 8, bt, d0 % 8 + 1, pl.ds(j * L, L)] = odd

    def issue_out(f, b):
        pltpu.async_copy(blks[b],
                         out_hbm.at[f, :, pl.ds(wid * BTW, BTW)],
                         sems[b])

    def drain_out(b):
        pltpu.make_async_copy(blks[b],
                              out_hbm.at[0, :, pl.ds(wid * BTW, BTW)],
                              sems[b]).wait()

    def f_loop(i, carry):
        for b in range(2):
            f = 2 * i + b

            @pl.when(i > 0)
            def _reuse():
                drain_out(b)

            wait_idx(b)
            fill_f(f, blks[b], idxb[b])

            @pl.when(i < F // 2 - 1)
            def _next():
                stage_idx(f + 2, b)

            issue_out(f, b)
        return carry

    lax.fori_loop(0, F // 2, f_loop, None)
    drain_out(0)
    drain_out(1)


def kernel(inputs, tables):
    idx3 = inputs.T.reshape(F, NW, BPW)
    bf = tables.transpose(0, 2, 1).astype(jnp.bfloat16)
    u16 = lax.bitcast_convert_type(bf, jnp.uint16).astype(jnp.uint32)
    packed = u16[:, 0::2, :] | (u16[:, 1::2, :] << 16)
    tt1 = lax.bitcast_convert_type(packed, jnp.int32).reshape(TT)
    out5 = _gather_kernel(idx3, tt1)
    return out5.transpose(2, 4, 0, 1, 3).reshape(B, F, D)


# speedup vs baseline: 1.0467x; 1.0467x over previous
import functools

import jax
import jax.numpy as jnp
from jax import lax
from jax.experimental import pallas as pl
from jax.experimental.pallas import tpu as pltpu
from jax.experimental.pallas import tpu_sc as plsc

B = 16384
F = 26
V1 = 51
D = 32

NC = 2
NS = 16
L = 16
NW = NC * NS

BPW = B // NW
DP = D // 2
DT = D // 8
BT = B // 128
BTW = BPW // 128
TT = F * DP * V1

_mesh = plsc.VectorSubcoreMesh(core_axis_name="c", subcore_axis_name="s")

_HI_MASK = -65536


@functools.partial(
    pl.kernel,
    mesh=_mesh,
    out_type=jax.ShapeDtypeStruct((F, DT, BT, 8, 128), jnp.float32),
    compiler_params=pltpu.CompilerParams(use_tc_tiling_on_sc=False,
                                         needs_layout_passes=False),
    scratch_types=[
        pltpu.VMEM((TT,), jnp.int32),
        pltpu.VMEM((F, BPW), jnp.int32),
        pltpu.VMEM((DT, BTW, 8, 128), jnp.float32),
        pltpu.VMEM((DT, BTW, 8, 128), jnp.float32),
        pltpu.SemaphoreType.DMA,
        pltpu.SemaphoreType.DMA,
    ],
)
def _gather_kernel(idx_hbm, tt_hbm, out_hbm, tt_v, idx_v, blk0, blk1,
                   osem0, osem1):
    wid = lax.axis_index("s") * NC + lax.axis_index("c")
    pltpu.sync_copy(tt_hbm, tt_v)
    pltpu.sync_copy(idx_hbm.at[:, wid], idx_v)

    blks = (blk0, blk1)
    sems = (osem0, osem1)

    def fill_f(f, blk):
        base0 = f * (DP * V1)

        @plsc.parallel_loop(0, BTW * 8, unroll=1)
        def _grp(k):
            bt = k // 8
            j = lax.rem(k, 8)
            idxv = idx_v[f, pl.ds(bt * 128 + j * L, L)]
            for dp in range(DP):
                base = base0 + dp * V1
                raw = plsc.load_gather(tt_v, [idxv + base])
                even = plsc.bitcast(raw << 16, jnp.float32)
                odd = plsc.bitcast(raw & _HI_MASK, jnp.float32)
                d0 = 2 * dp
                blk[d0 // 8, bt, d0 % 8, pl.ds(j * L, L)] = even
                blk[d0 // 8, bt, d0 % 8 + 1, pl.ds(j * L, L)] = odd

    def issue_out(f, b):
        pltpu.async_copy(blks[b],
                         out_hbm.at[f, :, pl.ds(wid * BTW, BTW)],
                         sems[b])

    def drain_out(b):
        pltpu.make_async_copy(blks[b],
                              out_hbm.at[0, :, pl.ds(wid * BTW, BTW)],
                              sems[b]).wait()

    def f_loop(i, carry):
        for b in range(2):
            f = 2 * i + b

            @pl.when(i > 0)
            def _reuse():
                drain_out(b)

            fill_f(f, blks[b])
            issue_out(f, b)
        return carry

    lax.fori_loop(0, F // 2, f_loop, None)
    drain_out(0)
    drain_out(1)


def kernel(inputs, tables):
    idx3 = inputs.T.reshape(F, NW, BPW)
    bf = tables.transpose(0, 2, 1).astype(jnp.bfloat16)
    u16 = lax.bitcast_convert_type(bf, jnp.uint16).astype(jnp.uint32)
    packed = u16[:, 0::2, :] | (u16[:, 1::2, :] << 16)
    tt1 = lax.bitcast_convert_type(packed, jnp.int32).reshape(TT)
    out5 = _gather_kernel(idx3, tt1)
    return out5.transpose(2, 4, 0, 1, 3).reshape(B, F, D)
